# Initial kernel scaffold; baseline (speedup 1.0000x reference)
import jax
import jax.numpy as jnp
from jax.experimental import pallas as pl


def kernel(feature_indices_0, feature_values_0, feature_indices_1, feature_values_1, weight, bias):
    raise NotImplementedError("write your pallas kernel here")


# speedup vs baseline: 7.0444x; 7.0444x over previous
import functools

import jax
import jax.numpy as jnp
from jax import lax
from jax.experimental import pallas as pl
from jax.experimental.pallas import tpu as pltpu
from jax.experimental.pallas import tpu_sc as plsc

NUM_OUTPUTS = 128
LANES = 16
NVREG = NUM_OUTPUTS // LANES
NUM_CORES = 2
NUM_SUBCORES = 16
NW = NUM_CORES * NUM_SUBCORES

CHUNK = 16
GATHER_SPLIT = 4


def _make_kernel(batch, max_active):
    rows_per_w = batch // NW
    nchunk = rows_per_w // CHUNK
    idx_per_chunk = CHUNK * max_active
    gwin = idx_per_chunk // GATHER_SPLIT
    assert idx_per_chunk % GATHER_SPLIT == 0
    assert batch % (NW * CHUNK) == 0

    mesh = plsc.VectorSubcoreMesh(core_axis_name="c", subcore_axis_name="s")
    out_sds = jax.ShapeDtypeStruct((batch, NUM_OUTPUTS), jnp.float32)

    @functools.partial(
        pl.kernel,
        out_type=(out_sds, out_sds),
        mesh=mesh,
        scratch_types=[
            pltpu.VMEM((GATHER_SPLIT, gwin), jnp.int32),
            pltpu.VMEM((CHUNK, 2 * LANES), jnp.float32),
            pltpu.VMEM((idx_per_chunk, NUM_OUTPUTS), jnp.float32),
            pltpu.VMEM((CHUNK, NUM_OUTPUTS), jnp.float32),
            pltpu.VMEM((NUM_OUTPUTS,), jnp.float32),
            pltpu.SemaphoreType.DMA,
        ],
    )
    def k(idx0_hbm, vals0_hbm, idx1_hbm, vals1_hbm, w_hbm, bias_hbm,
          out0_hbm, out1_hbm,
          idx_v, vals_v, rows_v, out_v, bias_v, sem):
        wid = lax.axis_index("s") * NUM_CORES + lax.axis_index("c")
        pltpu.sync_copy(bias_hbm, bias_v)

        for idx_hbm, vals_hbm, out_hbm in (
            (idx0_hbm, vals0_hbm, out0_hbm),
            (idx1_hbm, vals1_hbm, out1_hbm),
        ):
            @pl.loop(0, nchunk)
            def _(c):
                pltpu.sync_copy(idx_hbm.at[wid, c], idx_v)
                pltpu.sync_copy(vals_hbm.at[wid, c], vals_v)
                copies = [
                    pltpu.async_copy(
                        w_hbm.at[idx_v.at[g]],
                        rows_v.at[pl.ds(g * gwin, gwin)],
                        sem,
                    )
                    for g in range(GATHER_SPLIT)
                ]
                for cp in copies:
                    cp.wait()

                @pl.loop(0, CHUNK)
                def _(r):
                    acc = [bias_v[pl.ds(kk * LANES, LANES)] for kk in range(NVREG)]
                    v0 = vals_v[r, pl.ds(0, LANES)]
                    v1 = vals_v[r, pl.ds(LANES, LANES)]
                    rbase = r * max_active
                    for j in range(max_active):
                        s = v0[j] if j < LANES else v1[j - LANES]
                        v = jnp.broadcast_to(s, (LANES,))
                        for kk in range(NVREG):
                            acc[kk] = acc[kk] + v * rows_v[rbase + j,
                                                           pl.ds(kk * LANES, LANES)]
                    for kk in range(NVREG):
                        out_v[r, pl.ds(kk * LANES, LANES)] = acc[kk]

                pltpu.sync_copy(
                    out_v, out_hbm.at[pl.ds(wid * rows_per_w + c * CHUNK, CHUNK)])

    return k


def kernel(feature_indices_0, feature_values_0, feature_indices_1,
           feature_values_1, weight, bias):
    batch, max_active = feature_indices_0.shape
    rows_per_w = batch // NW
    nchunk = rows_per_w // CHUNK
    gwin = CHUNK * max_active // GATHER_SPLIT

    def shape_idx(a):
        return a.reshape(NW, nchunk, GATHER_SPLIT, gwin)

    def shape_vals(a):
        pad = jnp.zeros((batch, 2 * LANES - max_active), a.dtype)
        return jnp.concatenate([a, pad], axis=1).reshape(
            NW, nchunk, CHUNK, 2 * LANES)

    k = _make_kernel(batch, max_active)
    out0, out1 = k(
        shape_idx(feature_indices_0), shape_vals(feature_values_0),
        shape_idx(feature_indices_1), shape_vals(feature_values_1),
        weight, bias,
    )
    return (out0, out1)


# speedup vs baseline: 10.8665x; 1.5426x over previous
import functools

import jax
import jax.numpy as jnp
from jax import lax
from jax.experimental import pallas as pl
from jax.experimental.pallas import tpu as pltpu
from jax.experimental.pallas import tpu_sc as plsc

NUM_OUTPUTS = 128
LANES = 16
NVREG = NUM_OUTPUTS // LANES
NUM_CORES = 2
NUM_SUBCORES = 16
NW = NUM_CORES * NUM_SUBCORES

CHUNK = 16
GATHER_SPLIT = 4


def _make_kernel(batch, max_active):
    rows_per_w = batch // NW
    nchunk = rows_per_w // CHUNK
    idx_per_chunk = CHUNK * max_active
    gwin = idx_per_chunk // GATHER_SPLIT
    assert idx_per_chunk % GATHER_SPLIT == 0
    assert batch % (NW * CHUNK) == 0
    assert nchunk % 2 == 0

    mesh = plsc.VectorSubcoreMesh(core_axis_name="c", subcore_axis_name="s")
    out_sds = jax.ShapeDtypeStruct((batch, NUM_OUTPUTS), jnp.float32)
    row_buf = pltpu.VMEM((idx_per_chunk, NUM_OUTPUTS), jnp.float32)

    @functools.partial(
        pl.kernel,
        out_type=(out_sds, out_sds),
        mesh=mesh,
        scratch_types=[
            pltpu.VMEM((GATHER_SPLIT, gwin), jnp.int32),
            pltpu.VMEM((GATHER_SPLIT, gwin), jnp.int32),
            pltpu.VMEM((CHUNK, 2 * LANES), jnp.float32),
            pltpu.VMEM((CHUNK, 2 * LANES), jnp.float32),
            row_buf,
            row_buf,
            pltpu.VMEM((CHUNK, NUM_OUTPUTS), jnp.float32),
            pltpu.VMEM((CHUNK, NUM_OUTPUTS), jnp.float32),
            pltpu.VMEM((NUM_OUTPUTS,), jnp.float32),
            pltpu.SemaphoreType.DMA,
            pltpu.SemaphoreType.DMA,
            pltpu.SemaphoreType.DMA,
            pltpu.SemaphoreType.DMA,
        ],
    )
    def k(idx0_hbm, vals0_hbm, idx1_hbm, vals1_hbm, w_hbm, bias_hbm,
          out0_hbm, out1_hbm,
          idx_a, idx_b, vals_a, vals_b, rows_a, rows_b, out_a, out_b,
          bias_v, sem_ga, sem_gb, sem_oa, sem_ob):
        wid = lax.axis_index("s") * NUM_CORES + lax.axis_index("c")
        pltpu.sync_copy(bias_hbm, bias_v)

        def fire(idx_hbm, vals_hbm, c, idx_v, vals_v, rows_v, sem):
            pltpu.sync_copy(idx_hbm.at[wid, c], idx_v)
            pltpu.sync_copy(vals_hbm.at[wid, c], vals_v)
            for g in range(GATHER_SPLIT):
                pltpu.async_copy(
                    w_hbm.at[idx_v.at[g]],
                    rows_v.at[pl.ds(g * gwin, gwin)],
                    sem,
                )

        def drain_rows(rows_v, sem):
            pltpu.make_async_copy(
                w_hbm.at[pl.ds(0, idx_per_chunk)], rows_v, sem).wait()

        def drain_out(out_hbm, out_v, sem):
            pltpu.make_async_copy(out_hbm.at[pl.ds(0, CHUNK)], out_v, sem).wait()

        def compute(vals_v, rows_v, out_v, out_hbm, c, sem):
            @pl.loop(0, CHUNK)
            def _(r):
                acc = [bias_v[pl.ds(kk * LANES, LANES)] for kk in range(NVREG)]
                v0 = vals_v[r, pl.ds(0, LANES)]
                v1 = vals_v[r, pl.ds(LANES, LANES)]
                rbase = r * max_active
                for j in range(max_active):
                    s = v0[j] if j < LANES else v1[j - LANES]
                    v = jnp.broadcast_to(s, (LANES,))
                    for kk in range(NVREG):
                        acc[kk] = acc[kk] + v * rows_v[rbase + j,
                                                       pl.ds(kk * LANES, LANES)]
                for kk in range(NVREG):
                    out_v[r, pl.ds(kk * LANES, LANES)] = acc[kk]

            pltpu.async_copy(
                out_v, out_hbm.at[pl.ds(wid * rows_per_w + c * CHUNK, CHUNK)],
                sem)

        for idx_hbm, vals_hbm, out_hbm in (
            (idx0_hbm, vals0_hbm, out0_hbm),
            (idx1_hbm, vals1_hbm, out1_hbm),
        ):
            fire(idx_hbm, vals_hbm, 0, idx_a, vals_a, rows_a, sem_ga)

            @pl.loop(0, nchunk, step=2)
            def _(c):
                fire(idx_hbm, vals_hbm, c + 1, idx_b, vals_b, rows_b, sem_gb)
                drain_rows(rows_a, sem_ga)

                @pl.when(c > 0)
                def _():
                    drain_out(out_hbm, out_a, sem_oa)
                compute(vals_a, rows_a, out_a, out_hbm, c, sem_oa)

                @pl.when(c + 2 < nchunk)
                def _():
                    fire(idx_hbm, vals_hbm, c + 2, idx_a, vals_a, rows_a,
                         sem_ga)
                drain_rows(rows_b, sem_gb)

                @pl.when(c > 0)
                def _():
                    drain_out(out_hbm, out_b, sem_ob)
                compute(vals_b, rows_b, out_b, out_hbm, c + 1, sem_ob)

            drain_out(out_hbm, out_a, sem_oa)
            drain_out(out_hbm, out_b, sem_ob)

    return k


def kernel(feature_indices_0, feature_values_0, feature_indices_1,
           feature_values_1, weight, bias):
    batch, max_active = feature_indices_0.shape
    rows_per_w = batch // NW
    nchunk = rows_per_w // CHUNK
    gwin = CHUNK * max_active // GATHER_SPLIT

    def shape_idx(a):
        return a.reshape(NW, nchunk, GATHER_SPLIT, gwin)

    def shape_vals(a):
        pad = jnp.zeros((batch, 2 * LANES - max_active), a.dtype)
        return jnp.concatenate([a, pad], axis=1).reshape(
            NW, nchunk, CHUNK, 2 * LANES)

    k = _make_kernel(batch, max_active)
    out0, out1 = k(
        shape_idx(feature_indices_0), shape_vals(feature_values_0),
        shape_idx(feature_indices_1), shape_vals(feature_values_1),
        weight, bias,
    )
    return (out0, out1)


# speedup vs baseline: 13.6395x; 1.2552x over previous
import functools

import jax
import jax.numpy as jnp
from jax import lax
from jax.experimental import pallas as pl
from jax.experimental.pallas import tpu as pltpu
from jax.experimental.pallas import tpu_sc as plsc

NUM_OUTPUTS = 128
LANES = 16
NVREG = NUM_OUTPUTS // LANES
NUM_CORES = 2
NUM_SUBCORES = 16
NW = NUM_CORES * NUM_SUBCORES

CHUNK = 16
GATHER_SPLIT = 4


def _make_kernel(batch, max_active):
    rows_per_w = batch // NW
    nchunk = rows_per_w // CHUNK
    idx_per_chunk = CHUNK * max_active
    gwin = idx_per_chunk // GATHER_SPLIT
    assert idx_per_chunk % GATHER_SPLIT == 0
    assert batch % (NW * CHUNK) == 0
    assert nchunk % 2 == 0

    mesh = plsc.VectorSubcoreMesh(core_axis_name="c", subcore_axis_name="s")
    out_sds = jax.ShapeDtypeStruct((batch, NUM_OUTPUTS), jnp.float32)
    idx_buf = pltpu.VMEM((nchunk, GATHER_SPLIT, gwin), jnp.int32)
    vals_buf = pltpu.VMEM((CHUNK, 2 * LANES), jnp.float32)
    row_buf = pltpu.VMEM((idx_per_chunk, NUM_OUTPUTS), jnp.float32)
    out_buf = pltpu.VMEM((CHUNK, NUM_OUTPUTS), jnp.float32)

    @functools.partial(
        pl.kernel,
        out_type=(out_sds, out_sds),
        mesh=mesh,
        scratch_types=[
            idx_buf, idx_buf,
            vals_buf, vals_buf,
            row_buf, row_buf,
            out_buf, out_buf,
            pltpu.VMEM((NUM_OUTPUTS,), jnp.float32),
            pltpu.SemaphoreType.DMA,
            pltpu.SemaphoreType.DMA,
            pltpu.SemaphoreType.DMA,
            pltpu.SemaphoreType.DMA,
        ],
    )
    def k(idx0_hbm, vals0_hbm, idx1_hbm, vals1_hbm, w_hbm, bias_hbm,
          out0_hbm, out1_hbm,
          idx0_v, idx1_v, vals_a, vals_b, rows_a, rows_b, out_a, out_b,
          bias_v, sem_ga, sem_gb, sem_oa, sem_ob):
        wid = lax.axis_index("s") * NUM_CORES + lax.axis_index("c")
        idx_stage0 = pltpu.async_copy(idx0_hbm.at[wid], idx0_v, sem_ga)
        idx_stage1 = pltpu.async_copy(idx1_hbm.at[wid], idx1_v, sem_gb)
        pltpu.sync_copy(bias_hbm, bias_v)
        idx_stage0.wait()
        idx_stage1.wait()

        def fire(idx_v, vals_hbm, c, vals_v, rows_v, sem):
            pltpu.async_copy(vals_hbm.at[wid, c], vals_v, sem)
            for g in range(GATHER_SPLIT):
                pltpu.async_copy(
                    w_hbm.at[idx_v.at[c, g]],
                    rows_v.at[pl.ds(g * gwin, gwin)],
                    sem,
                )

        def drain_in(vals_hbm, vals_v, rows_v, sem):
            pltpu.make_async_copy(
                w_hbm.at[pl.ds(0, idx_per_chunk)], rows_v, sem).wait()
            pltpu.make_async_copy(vals_hbm.at[wid, 0], vals_v, sem).wait()

        def drain_out(out_hbm, out_v, sem):
            pltpu.make_async_copy(out_hbm.at[pl.ds(0, CHUNK)], out_v, sem).wait()

        def compute(vals_v, rows_v, out_v, out_hbm, c, sem):
            @pl.loop(0, CHUNK)
            def _(r):
                acc = [bias_v[pl.ds(kk * LANES, LANES)] for kk in range(NVREG)]
                v0 = vals_v[r, pl.ds(0, LANES)]
                v1 = vals_v[r, pl.ds(LANES, LANES)]
                rbase = r * max_active
                for j in range(max_active):
                    s = v0[j] if j < LANES else v1[j - LANES]
                    v = jnp.broadcast_to(s, (LANES,))
                    for kk in range(NVREG):
                        acc[kk] = acc[kk] + v * rows_v[rbase + j,
                                                       pl.ds(kk * LANES, LANES)]
                for kk in range(NVREG):
                    out_v[r, pl.ds(kk * LANES, LANES)] = acc[kk]

            pltpu.async_copy(
                out_v, out_hbm.at[pl.ds(wid * rows_per_w + c * CHUNK, CHUNK)],
                sem)

        for idx_v, vals_hbm, out_hbm in (
            (idx0_v, vals0_hbm, out0_hbm),
            (idx1_v, vals1_hbm, out1_hbm),
        ):
            fire(idx_v, vals_hbm, 0, vals_a, rows_a, sem_ga)

            @pl.loop(0, nchunk, step=2)
            def _(c):
                fire(idx_v, vals_hbm, c + 1, vals_b, rows_b, sem_gb)
                drain_in(vals_hbm, vals_a, rows_a, sem_ga)

                @pl.when(c > 0)
                def _():
                    drain_out(out_hbm, out_a, sem_oa)
                compute(vals_a, rows_a, out_a, out_hbm, c, sem_oa)

                @pl.when(c + 2 < nchunk)
                def _():
                    fire(idx_v, vals_hbm, c + 2, vals_a, rows_a, sem_ga)
                drain_in(vals_hbm, vals_b, rows_b, sem_gb)

                @pl.when(c > 0)
                def _():
                    drain_out(out_hbm, out_b, sem_ob)
                compute(vals_b, rows_b, out_b, out_hbm, c + 1, sem_ob)

            drain_out(out_hbm, out_a, sem_oa)
            drain_out(out_hbm, out_b, sem_ob)

    return k


def kernel(feature_indices_0, feature_values_0, feature_indices_1,
           feature_values_1, weight, bias):
    batch, max_active = feature_indices_0.shape
    rows_per_w = batch // NW
    nchunk = rows_per_w // CHUNK
    gwin = CHUNK * max_active // GATHER_SPLIT

    def shape_idx(a):
        return a.reshape(NW, nchunk, GATHER_SPLIT, gwin)

    def shape_vals(a):
        pad = jnp.zeros((batch, 2 * LANES - max_active), a.dtype)
        return jnp.concatenate([a, pad], axis=1).reshape(
            NW, nchunk, CHUNK, 2 * LANES)

    k = _make_kernel(batch, max_active)
    out0, out1 = k(
        shape_idx(feature_indices_0), shape_vals(feature_values_0),
        shape_idx(feature_indices_1), shape_vals(feature_values_1),
        weight, bias,
    )
    return (out0, out1)
